# Initial kernel scaffold; baseline (speedup 1.0000x reference)
import jax
import jax.numpy as jnp
from jax.experimental import pallas as pl


def kernel(emb, pos, sid, mod, role, padding_mask, W, bproj, cls_content, pos_tab, id_tab, mod_tab, role_tab):
    raise NotImplementedError("write your pallas kernel here")


# speedup vs baseline: 5.2913x; 5.2913x over previous
import jax
import jax.numpy as jnp
from jax import lax
from jax.experimental import pallas as pl
from jax.experimental.pallas import tpu as pltpu

_T = 256


def _body(sid_ref, pos_ref, mod_ref, role_ref, mask_ref, emb_ref, w_ref,
          bproj_ref, postab_ref, idtab_ref, mrtab_ref, out_ref):
    T = _T
    S = bproj_ref.shape[0]
    D = emb_ref.shape[1]
    PPAD = postab_ref.shape[0]

    sid = sid_ref[...]
    emb = emb_ref[...]

    embrep = pltpu.repeat(emb, S, axis=1)
    col = lax.broadcasted_iota(jnp.int32, (T, S * D), 1)
    X = jnp.where((col // D) == sid, embrep, jnp.bfloat16(0.0))
    acc = jnp.dot(X, w_ref[...], preferred_element_type=jnp.float32)

    scol = lax.broadcasted_iota(jnp.int32, (T, S), 1)
    oh_s = (scol == sid).astype(jnp.float32)
    acc = acc + jnp.dot(oh_s, bproj_ref[...], preferred_element_type=jnp.float32)

    acc = acc * mask_ref[...]

    pcol = lax.broadcasted_iota(jnp.int32, (T, PPAD), 1)
    oh_p = (pcol == pos_ref[...]).astype(jnp.float32)
    acc = acc + jnp.dot(oh_p, postab_ref[...], preferred_element_type=jnp.float32)

    acc = acc + jnp.dot(oh_s, idtab_ref[...], preferred_element_type=jnp.float32)

    mcol = lax.broadcasted_iota(jnp.int32, (T, 16), 1)
    oh_mr = ((mcol == mod_ref[...]) | (mcol == (role_ref[...] + 8))
             ).astype(jnp.float32)
    acc = acc + jnp.dot(oh_mr, mrtab_ref[...], preferred_element_type=jnp.float32)

    out_ref[...] = acc


def kernel(emb, pos, sid, mod, role, padding_mask, W, bproj, cls_content,
           pos_tab, id_tab, mod_tab, role_tab):
    B, L, D = emb.shape
    S, _, M = W.shape
    N = B * L
    T = _T
    G = N // T

    emb2 = emb.reshape(N, D).astype(jnp.bfloat16)
    sid2 = sid.reshape(N, 1).astype(jnp.int32)
    pos2 = pos.reshape(N, 1).astype(jnp.int32)
    mod2 = mod.reshape(N, 1).astype(jnp.int32)
    role2 = role.reshape(N, 1).astype(jnp.int32)
    mask2 = padding_mask.reshape(N, 1).astype(jnp.float32)

    w_flat = W.reshape(S * D, M).astype(jnp.bfloat16)
    bproj_f = bproj.astype(jnp.float32)

    P = pos_tab.shape[0]
    PPAD = ((P + 7) // 8) * 8
    postab_p = jnp.zeros((PPAD, M), jnp.float32).at[:P].set(pos_tab)
    idtab64 = id_tab[:S]
    mrtab = jnp.zeros((16, M), jnp.float32)
    mrtab = mrtab.at[:mod_tab.shape[0]].set(mod_tab)
    mrtab = mrtab.at[8:8 + role_tab.shape[0]].set(role_tab)

    tok_spec = lambda shp: pl.BlockSpec(shp, lambda i: (i, 0))
    full_spec = lambda shp: pl.BlockSpec(shp, lambda i: (0, 0))

    body_out = pl.pallas_call(
        _body,
        grid=(G,),
        in_specs=[
            tok_spec((T, 1)), tok_spec((T, 1)), tok_spec((T, 1)),
            tok_spec((T, 1)), tok_spec((T, 1)), tok_spec((T, D)),
            full_spec((S * D, M)), full_spec((S, M)), full_spec((PPAD, M)),
            full_spec((S, M)), full_spec((16, M)),
        ],
        out_specs=tok_spec((T, M)),
        out_shape=jax.ShapeDtypeStruct((N, M), jnp.float32),
        compiler_params=pltpu.CompilerParams(
            dimension_semantics=("parallel",)),
    )(sid2, pos2, mod2, role2, mask2, emb2, w_flat, bproj_f, postab_p,
      idtab64, mrtab)

    cls_row = cls_content + pos_tab[0] + id_tab[S]
    tokens = jnp.concatenate(
        [jnp.broadcast_to(cls_row, (B, 1, M)), body_out.reshape(B, L, M)],
        axis=1)
    attn_keep = jnp.concatenate(
        [jnp.ones((B, 1), dtype=bool), padding_mask], axis=1)
    return tokens, attn_keep
